# Initial kernel scaffold; baseline (speedup 1.0000x reference)
import jax
import jax.numpy as jnp
from jax.experimental import pallas as pl


def kernel(x, edge_index, graph_ids, eps, mlp_w, mlp_b, bn_gamma, bn_beta, lin_w, lin_b):
    raise NotImplementedError("write your pallas kernel here")


# speedup vs baseline: 3.7369x; 3.7369x over previous
import functools

import jax
import jax.numpy as jnp
from jax import lax
from jax.experimental import pallas as pl
from jax.experimental.pallas import tpu as pltpu
from jax.experimental.pallas import tpu_sc as plsc

_N = 10000
_E = 160000
_D = 256
_H = 128
_G = 32
_OUT = 128
_L = 4
_BN_EPS = 1e-3

_CHUNK = 128
_NCHUNKS = _E // _CHUNK
_SUBCORES = 16
_ROUNDS = (_NCHUNKS + _SUBCORES - 1) // _SUBCORES
_RPT = 624
_TAIL0 = _SUBCORES * _RPT
_TAIL = _N - _TAIL0

_R = 400
_NB = _N // _R


def _sc_agg_body(h_lo, h_hi, src, dst, out_lo, out_hi,
                 spmem, src_v, dst_v, rows_v, sem):
    c = lax.axis_index("c")
    s = lax.axis_index("s")

    def run(h_ref, out_ref):
        r0 = s * _RPT
        pltpu.sync_copy(h_ref.at[pl.ds(r0, _RPT)], spmem.at[pl.ds(r0, _RPT)])

        @pl.when(s == 0)
        def _():
            pltpu.sync_copy(h_ref.at[pl.ds(_TAIL0, _TAIL)],
                            spmem.at[pl.ds(_TAIL0, _TAIL)])

        plsc.subcore_barrier()

        def body(j, carry):
            chunk = j * _SUBCORES + s

            @pl.when(chunk < _NCHUNKS)
            def _():
                base = chunk * _CHUNK
                pltpu.sync_copy(src.at[pl.ds(base, _CHUNK)], src_v)
                pltpu.sync_copy(dst.at[pl.ds(base, _CHUNK)], dst_v)
                pltpu.async_copy(h_ref.at[src_v], rows_v, sem).wait()
                pltpu.sync_copy(rows_v, spmem.at[dst_v], add=True)

            return carry

        lax.fori_loop(0, _ROUNDS, body, 0)
        plsc.subcore_barrier()
        pltpu.sync_copy(spmem.at[pl.ds(r0, _RPT)], out_ref.at[pl.ds(r0, _RPT)])

        @pl.when(s == 0)
        def _():
            pltpu.sync_copy(spmem.at[pl.ds(_TAIL0, _TAIL)],
                            out_ref.at[pl.ds(_TAIL0, _TAIL)])

    @pl.when(c == 0)
    def _():
        run(h_lo, out_lo)

    @pl.when(c == 1)
    def _():
        run(h_hi, out_hi)


_sc_agg = pl.kernel(
    _sc_agg_body,
    out_type=(
        jax.ShapeDtypeStruct((_N, _H), jnp.float32),
        jax.ShapeDtypeStruct((_N, _H), jnp.float32),
    ),
    mesh=plsc.VectorSubcoreMesh(core_axis_name="c", subcore_axis_name="s"),
    scratch_types=[
        pltpu.VMEM_SHARED((_N, _H), jnp.float32),
        pltpu.VMEM((_CHUNK,), jnp.int32),
        pltpu.VMEM((_CHUNK,), jnp.int32),
        pltpu.VMEM((_CHUNK, _H), jnp.float32),
        pltpu.SemaphoreType.DMA,
    ],
)


def _mlp_body(eps_ref, slo_ref, shi_ref, hlo_ref, hhi_ref,
              w1_ref, b1_ref, w2_ref, b2_ref,
              u_ref, ssum_ref, ssq_ref):
    i = pl.program_id(0)
    eps_l = eps_ref[0]
    a_lo = slo_ref[...] + eps_l * hlo_ref[...]
    a_hi = shi_ref[...] + eps_l * hhi_ref[...]
    t = jnp.dot(a_lo, w1_ref[0:_H, :], preferred_element_type=jnp.float32)
    t = t + jnp.dot(a_hi, w1_ref[_H:_D, :], preferred_element_type=jnp.float32)
    t = jnp.maximum(t + b1_ref[...], 0.0)
    u = jnp.dot(t, w2_ref[...], preferred_element_type=jnp.float32) + b2_ref[...]
    u_ref[...] = u

    @pl.when(i == 0)
    def _():
        ssum_ref[...] = jnp.zeros_like(ssum_ref)
        ssq_ref[...] = jnp.zeros_like(ssq_ref)

    ssum_ref[...] += jnp.sum(u, axis=0, keepdims=True)
    ssq_ref[...] += jnp.sum(u * u, axis=0, keepdims=True)


_mlp_call = pl.pallas_call(
    _mlp_body,
    grid=(_NB,),
    in_specs=[
        pl.BlockSpec(memory_space=pltpu.SMEM),
        pl.BlockSpec((_R, _H), lambda i: (i, 0)),
        pl.BlockSpec((_R, _H), lambda i: (i, 0)),
        pl.BlockSpec((_R, _H), lambda i: (i, 0)),
        pl.BlockSpec((_R, _H), lambda i: (i, 0)),
        pl.BlockSpec((_D, _D), lambda i: (0, 0)),
        pl.BlockSpec((1, _D), lambda i: (0, 0)),
        pl.BlockSpec((_D, _D), lambda i: (0, 0)),
        pl.BlockSpec((1, _D), lambda i: (0, 0)),
    ],
    out_specs=[
        pl.BlockSpec((_R, _D), lambda i: (i, 0)),
        pl.BlockSpec((1, _D), lambda i: (0, 0)),
        pl.BlockSpec((1, _D), lambda i: (0, 0)),
    ],
    out_shape=[
        jax.ShapeDtypeStruct((_N, _D), jnp.float32),
        jax.ShapeDtypeStruct((1, _D), jnp.float32),
        jax.ShapeDtypeStruct((1, _D), jnp.float32),
    ],
)


def _onehot(ids):
    return (ids[:, None] == lax.broadcasted_iota(jnp.int32, (_R, _G), 1)
            ).astype(jnp.float32)


def _bn_body(gid_ref, u_ref, ssum_ref, ssq_ref, gam_ref, bet_ref,
             hlo_ref, hhi_ref, g_ref):
    i = pl.program_id(0)
    mean = ssum_ref[...] * (1.0 / _N)
    var = ssq_ref[...] * (1.0 / _N) - mean * mean
    scale = gam_ref[...] * lax.rsqrt(var + _BN_EPS)
    h = jnp.maximum((u_ref[...] - mean) * scale + bet_ref[...], 0.0)
    hlo_ref[...] = h[:, 0:_H]
    hhi_ref[...] = h[:, _H:_D]

    @pl.when(i == 0)
    def _():
        g_ref[...] = jnp.zeros_like(g_ref)

    oh = _onehot(gid_ref[0, 0])
    g_ref[...] += lax.dot_general(oh, h, (((0,), (0,)), ((), ())),
                                  preferred_element_type=jnp.float32)


_bn_call = pl.pallas_call(
    _bn_body,
    grid=(_NB,),
    in_specs=[
        pl.BlockSpec((1, 1, _R), lambda i: (i, 0, 0)),
        pl.BlockSpec((_R, _D), lambda i: (i, 0)),
        pl.BlockSpec((1, _D), lambda i: (0, 0)),
        pl.BlockSpec((1, _D), lambda i: (0, 0)),
        pl.BlockSpec((1, _D), lambda i: (0, 0)),
        pl.BlockSpec((1, _D), lambda i: (0, 0)),
    ],
    out_specs=[
        pl.BlockSpec((_R, _H), lambda i: (i, 0)),
        pl.BlockSpec((_R, _H), lambda i: (i, 0)),
        pl.BlockSpec((_G, _D), lambda i: (0, 0)),
    ],
    out_shape=[
        jax.ShapeDtypeStruct((_N, _H), jnp.float32),
        jax.ShapeDtypeStruct((_N, _H), jnp.float32),
        jax.ShapeDtypeStruct((_G, _D), jnp.float32),
    ],
)


def _seg_body(gid_ref, x_ref, g_ref):
    i = pl.program_id(0)

    @pl.when(i == 0)
    def _():
        g_ref[...] = jnp.zeros_like(g_ref)

    oh = _onehot(gid_ref[0, 0])
    g_ref[...] += lax.dot_general(oh, x_ref[...], (((0,), (0,)), ((), ())),
                                  preferred_element_type=jnp.float32)


_seg_call = pl.pallas_call(
    _seg_body,
    grid=(_NB,),
    in_specs=[
        pl.BlockSpec((1, 1, _R), lambda i: (i, 0, 0)),
        pl.BlockSpec((_R, _D), lambda i: (i, 0)),
    ],
    out_specs=pl.BlockSpec((_G, _D), lambda i: (0, 0)),
    out_shape=jax.ShapeDtypeStruct((_G, _D), jnp.float32),
)


def _readout_body(g_ref, w_ref, b_ref, out_ref):
    l = pl.program_id(0)

    @pl.when(l == 0)
    def _():
        out_ref[...] = jnp.zeros_like(out_ref)

    out_ref[...] += (jnp.dot(g_ref[0], w_ref[0],
                             preferred_element_type=jnp.float32) + b_ref[0, 0])


_readout_call = pl.pallas_call(
    _readout_body,
    grid=(_L + 1,),
    in_specs=[
        pl.BlockSpec((1, _G, _D), lambda l: (l, 0, 0)),
        pl.BlockSpec((1, _D, _OUT), lambda l: (l, 0, 0)),
        pl.BlockSpec((1, 1, _OUT), lambda l: (l, 0, 0)),
    ],
    out_specs=pl.BlockSpec((_G, _OUT), lambda l: (0, 0)),
    out_shape=jax.ShapeDtypeStruct((_G, _OUT), jnp.float32),
)


def kernel(x, edge_index, graph_ids, eps, mlp_w, mlp_b,
           bn_gamma, bn_beta, lin_w, lin_b):
    src = edge_index[0].astype(jnp.int32)
    dst = edge_index[1].astype(jnp.int32)
    gid3 = graph_ids.astype(jnp.int32).reshape(_NB, 1, _R)

    h_lo = x[:, :_H]
    h_hi = x[:, _H:]

    g_list = [_seg_call(gid3, x)]
    for layer in range(_L):
        s_lo, s_hi = _sc_agg(h_lo, h_hi, src, dst)
        u, ssum, ssq = _mlp_call(
            eps[layer].reshape(1), s_lo, s_hi, h_lo, h_hi,
            mlp_w[layer, 0], mlp_b[layer, 0].reshape(1, _D),
            mlp_w[layer, 1], mlp_b[layer, 1].reshape(1, _D))
        h_lo, h_hi, g = _bn_call(
            gid3, u, ssum, ssq,
            bn_gamma[layer].reshape(1, _D), bn_beta[layer].reshape(1, _D))
        g_list.append(g)

    g_all = jnp.stack(g_list)
    return _readout_call(g_all, lin_w, lin_b.reshape(_L + 1, 1, _OUT))
